# baseline (speedup 1.0000x reference)
import functools

import jax
import jax.numpy as jnp
from jax import lax
from jax.experimental import pallas as pl
from jax.experimental.pallas import tpu as pltpu
from jax.experimental.pallas import tpu_sc as plsc

_THRESH = 0.35667494393873245
_LB_IGNORE = 255
_FACTOR = 16

_NB, _NC, _H, _W = 8, 19, 512, 512
_NPIX = _NB * _H * _W
_K_STATIC = max(_NPIX // _FACTOR, 1)
_LN2 = 0.6931471805599453

_NROW = _NB * _H
_SC_ROWS = 1280
_TC_ROWS = _NROW - _SC_ROWS

_NWORK = 32
_ROWS_PER_WORK = _SC_ROWS // _NWORK
_CR = 8
_CW = 256
_CHUNK = _CR * _CW
_NSTRIPE = _ROWS_PER_WORK // _CR
_NHALF = _W // _CW
_NCHUNK = _NSTRIPE * _NHALF
_GROUPS = _CHUNK // 16

_TC_RB = 128
_TC_B0 = _SC_ROWS // _TC_RB
_TC_STEPS = _TC_ROWS // _TC_RB
_TC_PER_IMG = _H // _TC_RB


def _log_f32(s):
    bits = lax.bitcast_convert_type(s, jnp.int32)
    e = jnp.right_shift(bits, 23) - 127
    m = lax.bitcast_convert_type(
        jnp.bitwise_or(jnp.bitwise_and(bits, 0x7FFFFF), 0x3F800000),
        jnp.float32)
    big = m > 1.4142135623730951
    m = jnp.where(big, m * 0.5, m)
    ef = (e + jnp.where(big, 1, 0)).astype(jnp.float32)
    z = m - 1.0
    c8, c7, c6, c5, c4, c3, c2, c1, c0 = (
        7.0376836292e-2, -1.1514610310e-1, 1.1676998740e-1,
        -1.2420140846e-1, 1.4249322787e-1, -1.6668057665e-1,
        2.0000714765e-1, -2.4999993993e-1, 3.3333331174e-1)
    z2 = z * z
    z4 = z2 * z2
    b0 = c1 * z + c0
    b1 = c3 * z + c2
    b2 = c5 * z + c4
    b3 = c7 * z + c6
    d0 = b1 * z2 + b0
    d1 = b3 * z2 + b2
    poly = (c8 * z4 + d1) * z4 + d0
    r = z * z2 * poly - 0.5 * z2
    return z + r + ef * _LN2


def _tree(xs, op):
    xs = list(xs)
    while len(xs) > 1:
        nxt = [op(xs[i], xs[i + 1]) for i in range(0, len(xs) - 1, 2)]
        if len(xs) % 2:
            nxt.append(xs[-1])
        xs = nxt
    return xs[0]


def _sc_body(logits_hbm, labels_hbm, loss_hbm, cnt_out, sum_out, val_out,
             lbuf, labv, lossv, stage, sem, osem):
    cid = lax.axis_index("c")
    sid = lax.axis_index("s")
    wid = sid * 2 + cid
    grow0 = wid * _ROWS_PER_WORK

    zeros = jnp.zeros((16,), jnp.float32)

    def chunk_coords(k):
        stripe = k // _NHALF
        half = k % _NHALF
        gr = grow0 + stripe * _CR
        return gr // _H, gr % _H, half * _CW, gr

    def fire_chunk(k, slot):
        img, r0, w0, _ = chunk_coords(k)
        for c in range(_NC):
            src = logits_hbm.at[img, c, pl.ds(r0, _CR), pl.ds(w0, _CW)]
            pltpu.async_copy(
                src, lbuf.at[pl.ds((slot * _NC + c) * _CR, _CR), :], sem)
        pltpu.async_copy(labels_hbm.at[img, pl.ds(r0, _CR), pl.ds(w0, _CW)],
                         labv.at[pl.ds(slot * _CR, _CR), :], sem)

    def drain_chunk(slot):
        for c in range(_NC):
            pltpu.make_async_copy(
                logits_hbm.at[0, 0, pl.ds(0, _CR), pl.ds(0, _CW)],
                lbuf.at[pl.ds((slot * _NC + c) * _CR, _CR), :], sem).wait()
        pltpu.make_async_copy(
            labels_hbm.at[0, pl.ds(0, _CR), pl.ds(0, _CW)],
            labv.at[pl.ds(slot * _CR, _CR), :], sem).wait()

    def compute_chunk(k, slot, carry):
        def one_group(r, w, acc):
            cnt_h, sum_h, cnt_v = acc

            def zload(c):
                return lbuf[(slot * _NC + c) * _CR + r, pl.ds(w, 16)]

            lab = labv[slot * _CR + r, pl.ds(w, 16)]
            valid = lab != _LB_IGNORE
            labc = jnp.minimum(jnp.maximum(lab, 0), _NC - 1)
            z0 = zload(0)
            m = z0
            z_l = z0
            for c in range(1, _NC):
                zc = zload(c)
                m = jnp.maximum(m, zc)
                z_l = jnp.where(labc == c, zc, z_l)
            s = _tree([jnp.exp(zload(c) - m) for c in range(_NC)], jnp.add)
            loss = _log_f32(s) - (z_l - m)
            loss = jnp.where(valid, loss, 0.0)
            hard = loss > _THRESH
            cnt_h = cnt_h + jnp.where(hard, 1.0, 0.0)
            sum_h = sum_h + jnp.where(hard, loss, 0.0)
            cnt_v = cnt_v + jnp.where(valid, 1.0, 0.0)
            lossv[slot * _CR + r, pl.ds(w, 16)] = loss
            return (cnt_h, sum_h, cnt_v)

        def pair_group_body(q, acc):
            r = jnp.right_shift(q, 3)
            wb = jnp.bitwise_and(q, 7) * 32
            acc = one_group(r, pl.multiple_of(wb, 16), acc)
            acc = one_group(r, pl.multiple_of(wb + 16, 16), acc)
            return acc

        carry = lax.fori_loop(0, _GROUPS // 2, pair_group_body, carry)
        _, _, w0, gr = chunk_coords(k)
        pltpu.async_copy(lossv.at[pl.ds(slot * _CR, _CR), :],
                         loss_hbm.at[pl.ds(gr, _CR), pl.ds(w0, _CW)],
                         osem)
        return carry

    def drain_loss(slot):
        pltpu.make_async_copy(
            lossv.at[pl.ds(slot * _CR, _CR), :],
            loss_hbm.at[pl.ds(0, _CR), pl.ds(0, _CW)], osem).wait()

    fire_chunk(0, 0)
    fire_chunk(1, 1)

    def pair_body(j2, carry):
        for slot in (0, 1):
            k = j2 * 2 + slot
            drain_chunk(slot)

            @pl.when(k >= 2)
            def _():
                drain_loss(slot)

            carry = compute_chunk(k, slot, carry)

            @pl.when(k + 2 < _NCHUNK)
            def _():
                fire_chunk(k + 2, slot)
        return carry

    cnt_h, sum_h, cnt_v = lax.fori_loop(
        0, _NCHUNK // 2, pair_body, (zeros, zeros, zeros))
    drain_loss(0)
    drain_loss(1)

    stage[pl.ds(0, 16)] = cnt_h
    pltpu.sync_copy(stage, cnt_out.at[pl.ds(wid * 16, 16)])
    stage[pl.ds(0, 16)] = sum_h
    pltpu.sync_copy(stage, sum_out.at[pl.ds(wid * 16, 16)])
    stage[pl.ds(0, 16)] = cnt_v
    pltpu.sync_copy(stage, val_out.at[pl.ds(wid * 16, 16)])


def _sc_main(logits, labels):
    mesh = plsc.VectorSubcoreMesh(core_axis_name="c", subcore_axis_name="s")
    f = pl.kernel(
        _sc_body,
        mesh=mesh,
        out_type=[
            jax.ShapeDtypeStruct((_SC_ROWS, _W), jnp.float32),
            jax.ShapeDtypeStruct((_NWORK * 16,), jnp.float32),
            jax.ShapeDtypeStruct((_NWORK * 16,), jnp.float32),
            jax.ShapeDtypeStruct((_NWORK * 16,), jnp.float32),
        ],
        scratch_types=[
            pltpu.VMEM((2 * _NC * _CR, _CW), jnp.float32),
            pltpu.VMEM((2 * _CR, _CW), jnp.int32),
            pltpu.VMEM((2 * _CR, _CW), jnp.float32),
            pltpu.VMEM((16,), jnp.float32),
            pltpu.SemaphoreType.DMA,
            pltpu.SemaphoreType.DMA,
        ],
    )
    return f(logits, labels)


def _tc_body(x_ref, lab_ref, loss_ref, part_ref):
    x = x_ref[0]
    lab = lab_ref[0]
    m = jnp.max(x, axis=0)
    s = jnp.sum(jnp.exp(x - m[None]), axis=0)
    valid = lab != _LB_IGNORE
    labc = jnp.minimum(jnp.maximum(lab, 0), _NC - 1)
    z_l = x[0]
    for c in range(1, _NC):
        z_l = jnp.where(labc == c, x[c], z_l)
    loss = jnp.log(s) - (z_l - m)
    loss = jnp.where(valid, loss, 0.0)
    loss_ref[...] = loss
    hard = loss > _THRESH
    part_ref[0, 0, 0] = jnp.sum(hard.astype(jnp.float32))
    part_ref[0, 0, 1] = jnp.sum(jnp.where(hard, loss, 0.0))
    part_ref[0, 0, 2] = jnp.sum(valid.astype(jnp.float32))


def _tc_main(logits, labels):
    return pl.pallas_call(
        _tc_body,
        grid=(_TC_STEPS,),
        in_specs=[
            pl.BlockSpec((1, _NC, _TC_RB, _W),
                         lambda i: ((_TC_B0 + i) // _TC_PER_IMG, 0,
                                    (_TC_B0 + i) % _TC_PER_IMG, 0)),
            pl.BlockSpec((1, _TC_RB, _W),
                         lambda i: ((_TC_B0 + i) // _TC_PER_IMG,
                                    (_TC_B0 + i) % _TC_PER_IMG, 0)),
        ],
        out_specs=[
            pl.BlockSpec((_TC_RB, _W), lambda i: (i, 0)),
            pl.BlockSpec((1, 1, 3), lambda i: (i, 0, 0),
                         memory_space=pltpu.SMEM),
        ],
        out_shape=[
            jax.ShapeDtypeStruct((_TC_ROWS, _W), jnp.float32),
            jax.ShapeDtypeStruct((_TC_STEPS, 1, 3), jnp.float32),
        ],
    )(logits, labels)


def _topk_mean_body(x1_ref, x2_ref, o_ref):
    x1 = jnp.maximum(x1_ref[...], 0.0)
    x2 = jnp.maximum(x2_ref[...], 0.0)
    b1 = lax.bitcast_convert_type(x1, jnp.int32)
    b2 = lax.bitcast_convert_type(x2, jnp.int32)
    kf = jnp.float32(_K_STATIC)

    def step(i, cand):
        test = jnp.bitwise_or(cand, lax.shift_left(jnp.int32(1), 30 - i))
        cnt = (jnp.sum((b1 >= test).astype(jnp.float32))
               + jnp.sum((b2 >= test).astype(jnp.float32)))
        return jnp.where(cnt >= kf, test, cand)

    cand = lax.fori_loop(0, 31, step, jnp.int32(0))
    v = lax.bitcast_convert_type(cand, jnp.float32)
    g1 = x1 > v
    g2 = x2 > v
    cnt_gt = (jnp.sum(g1.astype(jnp.float32))
              + jnp.sum(g2.astype(jnp.float32)))
    sum_gt = (jnp.sum(jnp.where(g1, x1, 0.0))
              + jnp.sum(jnp.where(g2, x2, 0.0)))
    o_ref[0, 0] = (sum_gt + (kf - cnt_gt) * v) / kf


def _topk_mean(losses):
    loss_sc, loss_tc = losses
    out = pl.pallas_call(
        _topk_mean_body,
        out_shape=jax.ShapeDtypeStruct((1, 1), jnp.float32),
        out_specs=pl.BlockSpec(memory_space=pltpu.SMEM),
    )(loss_sc, loss_tc)
    return out[0, 0]


def kernel(logits, labels):
    loss_sc, cnt_h, sum_h, cnt_v = _sc_main(logits, labels)
    loss_tc, tc_part = _tc_main(logits, labels)
    n_hard_f = jnp.sum(cnt_h) + jnp.sum(tc_part[:, 0, 0])
    sum_hard = jnp.sum(sum_h) + jnp.sum(tc_part[:, 0, 1])
    n_valid_f = jnp.sum(cnt_v) + jnp.sum(tc_part[:, 0, 2])
    n_hard = n_hard_f.astype(jnp.int32)
    n_min = n_valid_f.astype(jnp.int32) // _FACTOR
    mean_hard = sum_hard / jnp.maximum(n_hard, 1).astype(jnp.float32)
    pred = n_hard < n_min
    return lax.cond(pred, _topk_mean, lambda _: mean_hard,
                    (loss_sc, loss_tc))


# speedup vs baseline: 1.1236x; 1.1236x over previous
import functools

import jax
import jax.numpy as jnp
from jax import lax
from jax.experimental import pallas as pl
from jax.experimental.pallas import tpu as pltpu
from jax.experimental.pallas import tpu_sc as plsc

_THRESH = 0.35667494393873245
_LB_IGNORE = 255
_FACTOR = 16

_NB, _NC, _H, _W = 8, 19, 512, 512
_NPIX = _NB * _H * _W
_K_STATIC = max(_NPIX // _FACTOR, 1)
_LN2 = 0.6931471805599453

_SCI = 2
_TCI = _NB - _SCI

_NWORK = 32
_WPI = _NWORK // _SCI
_ROWS_PER_WORK = _H // _WPI
_CR = 8
_CW = 256
_CHUNK = _CR * _CW
_NSTRIPE = _ROWS_PER_WORK // _CR
_NHALF = _W // _CW
_NCHUNK = _NSTRIPE * _NHALF
_GROUPS = _CHUNK // 16

_TC_RB = 128
_TC_NRB = _H // _TC_RB
_TC_STEPS = _TCI * _TC_NRB


def _log_f32(s):
    bits = lax.bitcast_convert_type(s, jnp.int32)
    e = jnp.right_shift(bits, 23) - 127
    m = lax.bitcast_convert_type(
        jnp.bitwise_or(jnp.bitwise_and(bits, 0x7FFFFF), 0x3F800000),
        jnp.float32)
    big = m > 1.4142135623730951
    m = jnp.where(big, m * 0.5, m)
    ef = (e + jnp.where(big, 1, 0)).astype(jnp.float32)
    z = m - 1.0
    c8, c7, c6, c5, c4, c3, c2, c1, c0 = (
        7.0376836292e-2, -1.1514610310e-1, 1.1676998740e-1,
        -1.2420140846e-1, 1.4249322787e-1, -1.6668057665e-1,
        2.0000714765e-1, -2.4999993993e-1, 3.3333331174e-1)
    z2 = z * z
    z4 = z2 * z2
    b0 = c1 * z + c0
    b1 = c3 * z + c2
    b2 = c5 * z + c4
    b3 = c7 * z + c6
    d0 = b1 * z2 + b0
    d1 = b3 * z2 + b2
    poly = (c8 * z4 + d1) * z4 + d0
    r = z * z2 * poly - 0.5 * z2
    return z + r + ef * _LN2


def _tree(xs, op):
    xs = list(xs)
    while len(xs) > 1:
        nxt = [op(xs[i], xs[i + 1]) for i in range(0, len(xs) - 1, 2)]
        if len(xs) % 2:
            nxt.append(xs[-1])
        xs = nxt
    return xs[0]


def _sc_body(logits_hbm, labels_hbm, loss_hbm, cnt_out, sum_out, val_out,
             lbuf, labv, lossv, stage, sem, osem):
    cid = lax.axis_index("c")
    sid = lax.axis_index("s")
    wid = sid * 2 + cid
    img = wid // _WPI
    row0 = (wid % _WPI) * _ROWS_PER_WORK

    zeros = jnp.zeros((16,), jnp.float32)

    def chunk_coords(k):
        stripe = k // _NHALF
        half = k % _NHALF
        return row0 + stripe * _CR, half * _CW

    def fire_chunk(k, slot):
        r0, w0 = chunk_coords(k)
        for c in range(_NC):
            src = logits_hbm.at[img, c, pl.ds(r0, _CR), pl.ds(w0, _CW)]
            pltpu.async_copy(
                src, lbuf.at[pl.ds((slot * _NC + c) * _CR, _CR), :], sem)
        pltpu.async_copy(labels_hbm.at[img, pl.ds(r0, _CR), pl.ds(w0, _CW)],
                         labv.at[pl.ds(slot * _CR, _CR), :], sem)

    def drain_chunk(slot):
        for c in range(_NC):
            pltpu.make_async_copy(
                logits_hbm.at[0, 0, pl.ds(0, _CR), pl.ds(0, _CW)],
                lbuf.at[pl.ds((slot * _NC + c) * _CR, _CR), :], sem).wait()
        pltpu.make_async_copy(
            labels_hbm.at[0, pl.ds(0, _CR), pl.ds(0, _CW)],
            labv.at[pl.ds(slot * _CR, _CR), :], sem).wait()

    def compute_chunk(k, slot, carry):
        def one_group(r, w, acc):
            cnt_h, sum_h, cnt_v = acc

            def zload(c):
                return lbuf[(slot * _NC + c) * _CR + r, pl.ds(w, 16)]

            lab = labv[slot * _CR + r, pl.ds(w, 16)]
            valid = lab != _LB_IGNORE
            labc = jnp.minimum(jnp.maximum(lab, 0), _NC - 1)
            z0 = zload(0)
            m = z0
            z_l = z0
            for c in range(1, _NC):
                zc = zload(c)
                m = jnp.maximum(m, zc)
                z_l = jnp.where(labc == c, zc, z_l)
            s = _tree([jnp.exp(zload(c) - m) for c in range(_NC)], jnp.add)
            loss = _log_f32(s) - (z_l - m)
            loss = jnp.where(valid, loss, 0.0)
            hard = loss > _THRESH
            cnt_h = cnt_h + jnp.where(hard, 1.0, 0.0)
            sum_h = sum_h + jnp.where(hard, loss, 0.0)
            cnt_v = cnt_v + jnp.where(valid, 1.0, 0.0)
            lossv[slot * _CR + r, pl.ds(w, 16)] = loss
            return (cnt_h, sum_h, cnt_v)

        def pair_group_body(q, acc):
            r = jnp.right_shift(q, 3)
            wb = jnp.bitwise_and(q, 7) * 32
            acc = one_group(r, pl.multiple_of(wb, 16), acc)
            acc = one_group(r, pl.multiple_of(wb + 16, 16), acc)
            return acc

        carry = lax.fori_loop(0, _GROUPS // 2, pair_group_body, carry)
        r0, w0 = chunk_coords(k)
        pltpu.async_copy(lossv.at[pl.ds(slot * _CR, _CR), :],
                         loss_hbm.at[img, pl.ds(r0, _CR), pl.ds(w0, _CW)],
                         osem)
        return carry

    def drain_loss(slot):
        pltpu.make_async_copy(
            lossv.at[pl.ds(slot * _CR, _CR), :],
            loss_hbm.at[0, pl.ds(0, _CR), pl.ds(0, _CW)], osem).wait()

    fire_chunk(0, 0)
    fire_chunk(1, 1)

    def pair_body(j2, carry):
        for slot in (0, 1):
            k = j2 * 2 + slot
            drain_chunk(slot)

            @pl.when(k >= 2)
            def _():
                drain_loss(slot)

            carry = compute_chunk(k, slot, carry)

            @pl.when(k + 2 < _NCHUNK)
            def _():
                fire_chunk(k + 2, slot)
        return carry

    cnt_h, sum_h, cnt_v = lax.fori_loop(
        0, _NCHUNK // 2, pair_body, (zeros, zeros, zeros))
    drain_loss(0)
    drain_loss(1)

    stage[pl.ds(0, 16)] = cnt_h
    pltpu.sync_copy(stage, cnt_out.at[pl.ds(wid * 16, 16)])
    stage[pl.ds(0, 16)] = sum_h
    pltpu.sync_copy(stage, sum_out.at[pl.ds(wid * 16, 16)])
    stage[pl.ds(0, 16)] = cnt_v
    pltpu.sync_copy(stage, val_out.at[pl.ds(wid * 16, 16)])


def _sc_main(logits, labels):
    mesh = plsc.VectorSubcoreMesh(core_axis_name="c", subcore_axis_name="s")
    f = pl.kernel(
        _sc_body,
        mesh=mesh,
        out_type=[
            jax.ShapeDtypeStruct((_SCI, _H, _W), jnp.float32),
            jax.ShapeDtypeStruct((_NWORK * 16,), jnp.float32),
            jax.ShapeDtypeStruct((_NWORK * 16,), jnp.float32),
            jax.ShapeDtypeStruct((_NWORK * 16,), jnp.float32),
        ],
        scratch_types=[
            pltpu.VMEM((2 * _NC * _CR, _CW), jnp.float32),
            pltpu.VMEM((2 * _CR, _CW), jnp.int32),
            pltpu.VMEM((2 * _CR, _CW), jnp.float32),
            pltpu.VMEM((16,), jnp.float32),
            pltpu.SemaphoreType.DMA,
            pltpu.SemaphoreType.DMA,
        ],
    )
    return f(logits, labels)


def _tc_body(x_ref, lab_ref, loss_ref, part_ref):
    x = x_ref[0]
    lab = lab_ref[0]
    m = jnp.max(x, axis=0)
    s = jnp.sum(jnp.exp(x - m[None]), axis=0)
    valid = lab != _LB_IGNORE
    labc = jnp.minimum(jnp.maximum(lab, 0), _NC - 1)
    z_l = x[0]
    for c in range(1, _NC):
        z_l = jnp.where(labc == c, x[c], z_l)
    loss = jnp.log(s) - (z_l - m)
    loss = jnp.where(valid, loss, 0.0)
    loss_ref[0] = loss
    hard = loss > _THRESH
    part_ref[0, 0, 0] = jnp.sum(hard.astype(jnp.float32))
    part_ref[0, 0, 1] = jnp.sum(jnp.where(hard, loss, 0.0))
    part_ref[0, 0, 2] = jnp.sum(valid.astype(jnp.float32))


def _tc_main(logits, labels):
    return pl.pallas_call(
        _tc_body,
        grid=(_TCI, _TC_NRB),
        in_specs=[
            pl.BlockSpec((1, _NC, _TC_RB, _W),
                         lambda j, i: (j + _SCI, 0, i, 0)),
            pl.BlockSpec((1, _TC_RB, _W), lambda j, i: (j + _SCI, i, 0)),
        ],
        out_specs=[
            pl.BlockSpec((1, _TC_RB, _W), lambda j, i: (j, i, 0)),
            pl.BlockSpec((1, 1, 3), lambda j, i: (j * _TC_NRB + i, 0, 0),
                         memory_space=pltpu.SMEM),
        ],
        out_shape=[
            jax.ShapeDtypeStruct((_TCI, _H, _W), jnp.float32),
            jax.ShapeDtypeStruct((_TC_STEPS, 1, 3), jnp.float32),
        ],
    )(logits, labels)


def _topk_mean_body(x1_ref, x2_ref, o_ref):
    x1 = jnp.maximum(x1_ref[...], 0.0)
    x2 = jnp.maximum(x2_ref[...], 0.0)
    b1 = lax.bitcast_convert_type(x1, jnp.int32)
    b2 = lax.bitcast_convert_type(x2, jnp.int32)
    kf = jnp.float32(_K_STATIC)

    def step(i, cand):
        test = jnp.bitwise_or(cand, lax.shift_left(jnp.int32(1), 30 - i))
        cnt = (jnp.sum((b1 >= test).astype(jnp.float32))
               + jnp.sum((b2 >= test).astype(jnp.float32)))
        return jnp.where(cnt >= kf, test, cand)

    cand = lax.fori_loop(0, 31, step, jnp.int32(0))
    v = lax.bitcast_convert_type(cand, jnp.float32)
    g1 = x1 > v
    g2 = x2 > v
    cnt_gt = (jnp.sum(g1.astype(jnp.float32))
              + jnp.sum(g2.astype(jnp.float32)))
    sum_gt = (jnp.sum(jnp.where(g1, x1, 0.0))
              + jnp.sum(jnp.where(g2, x2, 0.0)))
    o_ref[0, 0] = (sum_gt + (kf - cnt_gt) * v) / kf


def _topk_mean(losses):
    loss_sc, loss_tc = losses
    out = pl.pallas_call(
        _topk_mean_body,
        out_shape=jax.ShapeDtypeStruct((1, 1), jnp.float32),
        out_specs=pl.BlockSpec(memory_space=pltpu.SMEM),
    )(loss_sc, loss_tc)
    return out[0, 0]


def kernel(logits, labels):
    loss_sc, cnt_h, sum_h, cnt_v = _sc_main(logits, labels)
    loss_tc, tc_part = _tc_main(logits, labels)
    n_hard_f = jnp.sum(cnt_h) + jnp.sum(tc_part[:, 0, 0])
    sum_hard = jnp.sum(sum_h) + jnp.sum(tc_part[:, 0, 1])
    n_valid_f = jnp.sum(cnt_v) + jnp.sum(tc_part[:, 0, 2])
    n_hard = n_hard_f.astype(jnp.int32)
    n_min = n_valid_f.astype(jnp.int32) // _FACTOR
    mean_hard = sum_hard / jnp.maximum(n_hard, 1).astype(jnp.float32)
    pred = n_hard < n_min
    return lax.cond(pred, _topk_mean, lambda _: mean_hard,
                    (loss_sc, loss_tc))
